# baseline (speedup 1.0000x reference)
import functools

import jax
import jax.numpy as jnp
from jax import lax
from jax.experimental import pallas as pl
from jax.experimental.pallas import tpu as pltpu
from jax.experimental.pallas import tpu_sc as plsc

_HIDDEN = 1024
_NUM_OPS = 8
_N_TOK = 4096
_BLK = 256
_NB = _N_TOK // _BLK + _NUM_OPS
_NPAD = _NB * _BLK


@functools.lru_cache(maxsize=None)
def _make_sc_gather(n_out: int, n_table: int):
    info = plsc.get_sparse_core_info()
    n_workers = info.num_cores * info.num_subcores
    rows_per_w = n_out // n_workers
    chunk = 64
    n_chunks = rows_per_w // chunk
    assert rows_per_w % chunk == 0 and n_out % n_workers == 0

    mesh = plsc.VectorSubcoreMesh(core_axis_name="c", subcore_axis_name="s")

    @functools.partial(
        pl.kernel,
        mesh=mesh,
        out_type=jax.ShapeDtypeStruct((n_out, _HIDDEN), jnp.float32),
        scratch_types=[
            pltpu.VMEM((chunk,), jnp.int32),
            pltpu.VMEM((chunk, _HIDDEN), jnp.float32),
            pltpu.SemaphoreType.DMA,
        ],
    )
    def gather(table_hbm, idx_hbm, out_hbm, idx_v, rows_v, sem):
        wid = lax.axis_index("s") * info.num_cores + lax.axis_index("c")
        base = wid * rows_per_w
        for c in range(n_chunks):
            r0 = base + c * chunk
            pltpu.sync_copy(idx_hbm.at[pl.ds(r0, chunk)], idx_v)
            pltpu.async_copy(table_hbm.at[idx_v], rows_v, sem).wait()
            pltpu.sync_copy(rows_v, out_hbm.at[pl.ds(r0, chunk)])

    return gather


def _gather_rows(table, idx):
    return _make_sc_gather(idx.shape[0], table.shape[0])(table, idx)


@functools.lru_cache(maxsize=None)
def _make_sc_scatter(n_in: int, n_out: int):
    info = plsc.get_sparse_core_info()
    n_workers = info.num_cores * info.num_subcores
    rows_per_w = n_in // n_workers
    chunk = 64
    n_chunks = rows_per_w // chunk
    assert rows_per_w % chunk == 0 and n_in % n_workers == 0

    mesh = plsc.VectorSubcoreMesh(core_axis_name="c", subcore_axis_name="s")

    @functools.partial(
        pl.kernel,
        mesh=mesh,
        out_type=jax.ShapeDtypeStruct((n_out, _HIDDEN), jnp.float32),
        scratch_types=[
            pltpu.VMEM((chunk,), jnp.int32),
            pltpu.VMEM((chunk, _HIDDEN), jnp.float32),
            pltpu.SemaphoreType.DMA,
        ],
    )
    def scatter(x_hbm, idx_hbm, out_hbm, idx_v, rows_v, sem):
        wid = lax.axis_index("s") * info.num_cores + lax.axis_index("c")
        base = wid * rows_per_w
        for c in range(n_chunks):
            r0 = base + c * chunk
            pltpu.sync_copy(idx_hbm.at[pl.ds(r0, chunk)], idx_v)
            pltpu.sync_copy(x_hbm.at[pl.ds(r0, chunk)], rows_v)
            pltpu.async_copy(rows_v, out_hbm.at[idx_v], sem).wait()

    return scatter


def _scatter_rows(x, idx, n_out):
    return _make_sc_scatter(x.shape[0], n_out)(x, idx)


def _mlp_body(meta_ref, x_ref, w1_ref, b1_ref, w2_ref, b2_ref, o_ref):
    b = pl.program_id(0)

    @pl.when(b < meta_ref[_NB])
    def _():
        xb = x_ref[...].astype(jnp.bfloat16)
        h = jnp.dot(xb, w1_ref[0], preferred_element_type=jnp.float32)
        h = jnp.maximum(h + b1_ref[0], 0.0)
        y = jnp.dot(h.astype(jnp.bfloat16), w2_ref[0],
                    preferred_element_type=jnp.float32)
        o_ref[...] = jnp.maximum(y + b2_ref[0], 0.0)


def _grouped_mlp(x_pad, W1, b1, W2, b2, meta):
    def _xo(b, m):
        return (jnp.minimum(b, m[_NB] - 1), 0)

    grid_spec = pltpu.PrefetchScalarGridSpec(
        num_scalar_prefetch=1,
        grid=(_NB,),
        in_specs=[
            pl.BlockSpec((_BLK, _HIDDEN), _xo),
            pl.BlockSpec((1, _HIDDEN, _HIDDEN), lambda b, m: (m[b], 0, 0)),
            pl.BlockSpec((1, 1, _HIDDEN), lambda b, m: (m[b], 0, 0)),
            pl.BlockSpec((1, _HIDDEN, _HIDDEN), lambda b, m: (m[b], 0, 0)),
            pl.BlockSpec((1, 1, _HIDDEN), lambda b, m: (m[b], 0, 0)),
        ],
        out_specs=pl.BlockSpec((_BLK, _HIDDEN), _xo),
    )
    return pl.pallas_call(
        _mlp_body,
        grid_spec=grid_spec,
        out_shape=jax.ShapeDtypeStruct((_NPAD, _HIDDEN), jnp.float32),
    )(meta, x_pad, W1.astype(jnp.bfloat16), b1[:, None, :],
      W2.astype(jnp.bfloat16), b2[:, None, :])


def _routing(chosen_ops):
    chosen = chosen_ops.astype(jnp.int32)
    onehot = (chosen[:, None] == jnp.arange(_NUM_OPS, dtype=jnp.int32)[None, :])
    counts = jnp.sum(onehot, axis=0, dtype=jnp.int32)
    nblk = (counts + _BLK - 1) // _BLK
    blk_end = jnp.cumsum(nblk)
    blk_start = jnp.concatenate([jnp.zeros(1, jnp.int32), blk_end[:-1]])
    num_active = blk_end[-1]
    oh32 = onehot.astype(jnp.int32)
    incl = jnp.cumsum(oh32, axis=0)
    rank = jnp.sum(incl * oh32, axis=1) - 1
    slot = blk_start[chosen] * _BLK + rank
    bids = jnp.arange(_NB, dtype=jnp.int32)
    eb = jnp.searchsorted(blk_end, bids, side="right").astype(jnp.int32)
    last_e = eb[jnp.maximum(num_active - 1, 0)]
    eb = jnp.where(bids < num_active, jnp.minimum(eb, _NUM_OPS - 1), last_e)
    meta = jnp.concatenate([eb, num_active[None]]).astype(jnp.int32)
    return slot, meta


def kernel(x, W1, b1, W2, b2, chosen_ops):
    slot, meta = _routing(chosen_ops)
    x_pad = _scatter_rows(x, slot, _NPAD)
    y_pad = _grouped_mlp(x_pad, W1, b1, W2, b2, meta)
    return _gather_rows(y_pad, slot)


# speedup vs baseline: 1.2219x; 1.2219x over previous
import functools

import jax
import jax.numpy as jnp
from jax import lax
from jax.experimental import pallas as pl
from jax.experimental.pallas import tpu as pltpu
from jax.experimental.pallas import tpu_sc as plsc

_HIDDEN = 1024
_NUM_OPS = 8
_N_TOK = 4096
_BLK = 256
_NB = _N_TOK // _BLK + _NUM_OPS
_NPAD = _NB * _BLK


@functools.lru_cache(maxsize=None)
def _make_sc_gather(n_out: int, n_table: int):
    info = plsc.get_sparse_core_info()
    n_workers = info.num_cores * info.num_subcores
    rows_per_w = n_out // n_workers
    chunk = 64
    n_chunks = rows_per_w // chunk
    assert rows_per_w % chunk == 0 and n_out % n_workers == 0

    mesh = plsc.VectorSubcoreMesh(core_axis_name="c", subcore_axis_name="s")

    @functools.partial(
        pl.kernel,
        mesh=mesh,
        out_type=jax.ShapeDtypeStruct((n_out, _HIDDEN), jnp.float32),
        scratch_types=[
            pltpu.VMEM((chunk,), jnp.int32),
            pltpu.VMEM((chunk, _HIDDEN), jnp.float32),
            pltpu.SemaphoreType.DMA,
        ],
    )
    def gather(table_hbm, idx_hbm, out_hbm, idx_v, rows_v, sem):
        wid = lax.axis_index("s") * info.num_cores + lax.axis_index("c")
        base = wid * rows_per_w
        for c in range(n_chunks):
            r0 = base + c * chunk
            pltpu.sync_copy(idx_hbm.at[pl.ds(r0, chunk)], idx_v)
            pltpu.async_copy(table_hbm.at[idx_v], rows_v, sem).wait()
            pltpu.sync_copy(rows_v, out_hbm.at[pl.ds(r0, chunk)])

    return gather


def _gather_rows(table, idx):
    return _make_sc_gather(idx.shape[0], table.shape[0])(table, idx)


@functools.lru_cache(maxsize=None)
def _make_sc_scatter(n_in: int, n_out: int):
    info = plsc.get_sparse_core_info()
    n_workers = info.num_cores * info.num_subcores
    rows_per_w = n_in // n_workers
    chunk = 64
    n_chunks = rows_per_w // chunk
    assert rows_per_w % chunk == 0 and n_in % n_workers == 0

    mesh = plsc.VectorSubcoreMesh(core_axis_name="c", subcore_axis_name="s")

    @functools.partial(
        pl.kernel,
        mesh=mesh,
        out_type=jax.ShapeDtypeStruct((n_out, _HIDDEN), jnp.float32),
        scratch_types=[
            pltpu.VMEM((chunk,), jnp.int32),
            pltpu.VMEM((chunk, _HIDDEN), jnp.float32),
            pltpu.SemaphoreType.DMA,
        ],
    )
    def scatter(x_hbm, idx_hbm, out_hbm, idx_v, rows_v, sem):
        wid = lax.axis_index("s") * info.num_cores + lax.axis_index("c")
        base = wid * rows_per_w
        for c in range(n_chunks):
            r0 = base + c * chunk
            pltpu.sync_copy(idx_hbm.at[pl.ds(r0, chunk)], idx_v)
            pltpu.sync_copy(x_hbm.at[pl.ds(r0, chunk)], rows_v)
            pltpu.async_copy(rows_v, out_hbm.at[idx_v], sem).wait()

    return scatter


def _scatter_rows(x, idx, n_out):
    return _make_sc_scatter(x.shape[0], n_out)(x, idx)


def _mlp_body(meta_ref, x_ref, w1_ref, b1_ref, w2_ref, b2_ref, o_ref):
    b = pl.program_id(0)

    @pl.when(b < meta_ref[_NB])
    def _():
        h = jnp.dot(x_ref[...], w1_ref[0], preferred_element_type=jnp.float32)
        h = jnp.maximum(h + b1_ref[0], 0.0)
        y = jnp.dot(h, w2_ref[0], preferred_element_type=jnp.float32)
        o_ref[...] = jnp.maximum(y + b2_ref[0], 0.0)


def _grouped_mlp(x_pad, W1, b1, W2, b2, meta):
    def _xo(b, m):
        return (jnp.minimum(b, m[_NB] - 1), 0)

    grid_spec = pltpu.PrefetchScalarGridSpec(
        num_scalar_prefetch=1,
        grid=(_NB,),
        in_specs=[
            pl.BlockSpec((_BLK, _HIDDEN), _xo),
            pl.BlockSpec((1, _HIDDEN, _HIDDEN), lambda b, m: (m[b], 0, 0)),
            pl.BlockSpec((1, 1, _HIDDEN), lambda b, m: (m[b], 0, 0)),
            pl.BlockSpec((1, _HIDDEN, _HIDDEN), lambda b, m: (m[b], 0, 0)),
            pl.BlockSpec((1, 1, _HIDDEN), lambda b, m: (m[b], 0, 0)),
        ],
        out_specs=pl.BlockSpec((_BLK, _HIDDEN), _xo),
    )
    return pl.pallas_call(
        _mlp_body,
        grid_spec=grid_spec,
        out_shape=jax.ShapeDtypeStruct((_NPAD, _HIDDEN), jnp.float32),
    )(meta, x_pad, W1, b1[:, None, :], W2, b2[:, None, :])


def _routing(chosen_ops):
    chosen = chosen_ops.astype(jnp.int32)
    onehot = (chosen[:, None] == jnp.arange(_NUM_OPS, dtype=jnp.int32)[None, :])
    counts = jnp.sum(onehot, axis=0, dtype=jnp.int32)
    nblk = (counts + _BLK - 1) // _BLK
    blk_end = jnp.cumsum(nblk)
    blk_start = jnp.concatenate([jnp.zeros(1, jnp.int32), blk_end[:-1]])
    num_active = blk_end[-1]
    oh32 = onehot.astype(jnp.int32)
    incl = jnp.cumsum(oh32, axis=0)
    rank = jnp.sum(incl * oh32, axis=1) - 1
    slot = blk_start[chosen] * _BLK + rank
    bids = jnp.arange(_NB, dtype=jnp.int32)
    eb = jnp.searchsorted(blk_end, bids, side="right").astype(jnp.int32)
    last_e = eb[jnp.maximum(num_active - 1, 0)]
    eb = jnp.where(bids < num_active, jnp.minimum(eb, _NUM_OPS - 1), last_e)
    meta = jnp.concatenate([eb, num_active[None]]).astype(jnp.int32)
    return slot, meta


def kernel(x, W1, b1, W2, b2, chosen_ops):
    slot, meta = _routing(chosen_ops)
    x_pad = _scatter_rows(x, slot, _NPAD)
    y_pad = _grouped_mlp(x_pad, W1, b1, W2, b2, meta)
    return _gather_rows(y_pad, slot)


# speedup vs baseline: 1.2679x; 1.0376x over previous
import functools

import jax
import jax.numpy as jnp
from jax import lax
from jax.experimental import pallas as pl
from jax.experimental.pallas import tpu as pltpu
from jax.experimental.pallas import tpu_sc as plsc

_HIDDEN = 1024
_NUM_OPS = 8
_N_TOK = 4096
_BLK = 512
_NB = _N_TOK // _BLK + _NUM_OPS
_NPAD = _NB * _BLK


@functools.lru_cache(maxsize=None)
def _make_sc_gather(n_out: int, n_table: int):
    info = plsc.get_sparse_core_info()
    n_workers = info.num_cores * info.num_subcores
    rows_per_w = n_out // n_workers
    chunk = 64
    n_chunks = rows_per_w // chunk
    assert rows_per_w % chunk == 0 and n_out % n_workers == 0

    mesh = plsc.VectorSubcoreMesh(core_axis_name="c", subcore_axis_name="s")

    @functools.partial(
        pl.kernel,
        mesh=mesh,
        out_type=jax.ShapeDtypeStruct((n_out, _HIDDEN), jnp.float32),
        scratch_types=[
            pltpu.VMEM((chunk,), jnp.int32),
            pltpu.VMEM((chunk, _HIDDEN), jnp.float32),
            pltpu.SemaphoreType.DMA,
        ],
    )
    def gather(table_hbm, idx_hbm, out_hbm, idx_v, rows_v, sem):
        wid = lax.axis_index("s") * info.num_cores + lax.axis_index("c")
        base = wid * rows_per_w
        for c in range(n_chunks):
            r0 = base + c * chunk
            pltpu.sync_copy(idx_hbm.at[pl.ds(r0, chunk)], idx_v)
            pltpu.async_copy(table_hbm.at[idx_v], rows_v, sem).wait()
            pltpu.sync_copy(rows_v, out_hbm.at[pl.ds(r0, chunk)])

    return gather


def _gather_rows(table, idx):
    return _make_sc_gather(idx.shape[0], table.shape[0])(table, idx)


@functools.lru_cache(maxsize=None)
def _make_sc_scatter(n_in: int, n_out: int):
    info = plsc.get_sparse_core_info()
    n_workers = info.num_cores * info.num_subcores
    rows_per_w = n_in // n_workers
    chunk = 64
    n_chunks = rows_per_w // chunk
    assert rows_per_w % chunk == 0 and n_in % n_workers == 0

    mesh = plsc.VectorSubcoreMesh(core_axis_name="c", subcore_axis_name="s")

    @functools.partial(
        pl.kernel,
        mesh=mesh,
        out_type=jax.ShapeDtypeStruct((n_out, _HIDDEN), jnp.float32),
        scratch_types=[
            pltpu.VMEM((chunk,), jnp.int32),
            pltpu.VMEM((chunk, _HIDDEN), jnp.float32),
            pltpu.SemaphoreType.DMA,
        ],
    )
    def scatter(x_hbm, idx_hbm, out_hbm, idx_v, rows_v, sem):
        wid = lax.axis_index("s") * info.num_cores + lax.axis_index("c")
        base = wid * rows_per_w
        for c in range(n_chunks):
            r0 = base + c * chunk
            pltpu.sync_copy(idx_hbm.at[pl.ds(r0, chunk)], idx_v)
            pltpu.sync_copy(x_hbm.at[pl.ds(r0, chunk)], rows_v)
            pltpu.async_copy(rows_v, out_hbm.at[idx_v], sem).wait()

    return scatter


def _scatter_rows(x, idx, n_out):
    return _make_sc_scatter(x.shape[0], n_out)(x, idx)


def _mlp_body(meta_ref, x_ref, w1_ref, b1_ref, w2_ref, b2_ref, o_ref):
    b = pl.program_id(0)

    @pl.when(b < meta_ref[_NB])
    def _():
        h = jnp.dot(x_ref[...], w1_ref[0], preferred_element_type=jnp.float32)
        h = jnp.maximum(h + b1_ref[0], 0.0)
        y = jnp.dot(h, w2_ref[0], preferred_element_type=jnp.float32)
        o_ref[...] = jnp.maximum(y + b2_ref[0], 0.0)


def _grouped_mlp(x_pad, W1, b1, W2, b2, meta):
    def _xo(b, m):
        return (jnp.minimum(b, m[_NB] - 1), 0)

    grid_spec = pltpu.PrefetchScalarGridSpec(
        num_scalar_prefetch=1,
        grid=(_NB,),
        in_specs=[
            pl.BlockSpec((_BLK, _HIDDEN), _xo),
            pl.BlockSpec((1, _HIDDEN, _HIDDEN), lambda b, m: (m[b], 0, 0)),
            pl.BlockSpec((1, 1, _HIDDEN), lambda b, m: (m[b], 0, 0)),
            pl.BlockSpec((1, _HIDDEN, _HIDDEN), lambda b, m: (m[b], 0, 0)),
            pl.BlockSpec((1, 1, _HIDDEN), lambda b, m: (m[b], 0, 0)),
        ],
        out_specs=pl.BlockSpec((_BLK, _HIDDEN), _xo),
    )
    return pl.pallas_call(
        _mlp_body,
        grid_spec=grid_spec,
        out_shape=jax.ShapeDtypeStruct((_NPAD, _HIDDEN), jnp.float32),
    )(meta, x_pad, W1, b1[:, None, :], W2, b2[:, None, :])


def _routing(chosen_ops):
    chosen = chosen_ops.astype(jnp.int32)
    onehot = (chosen[:, None] == jnp.arange(_NUM_OPS, dtype=jnp.int32)[None, :])
    counts = jnp.sum(onehot, axis=0, dtype=jnp.int32)
    nblk = (counts + _BLK - 1) // _BLK
    blk_end = jnp.cumsum(nblk)
    blk_start = jnp.concatenate([jnp.zeros(1, jnp.int32), blk_end[:-1]])
    num_active = blk_end[-1]
    oh32 = onehot.astype(jnp.int32)
    incl = jnp.cumsum(oh32, axis=0)
    rank = jnp.sum(incl * oh32, axis=1) - 1
    slot = blk_start[chosen] * _BLK + rank
    bids = jnp.arange(_NB, dtype=jnp.int32)
    eb = jnp.searchsorted(blk_end, bids, side="right").astype(jnp.int32)
    last_e = eb[jnp.maximum(num_active - 1, 0)]
    eb = jnp.where(bids < num_active, jnp.minimum(eb, _NUM_OPS - 1), last_e)
    meta = jnp.concatenate([eb, num_active[None]]).astype(jnp.int32)
    return slot, meta


def kernel(x, W1, b1, W2, b2, chosen_ops):
    slot, meta = _routing(chosen_ops)
    x_pad = _scatter_rows(x, slot, _NPAD)
    y_pad = _grouped_mlp(x_pad, W1, b1, W2, b2, meta)
    return _gather_rows(y_pad, slot)
